# Initial kernel scaffold; baseline (speedup 1.0000x reference)
import jax
import jax.numpy as jnp
from jax.experimental import pallas as pl


def kernel(x, rbf_ij, sph_ij, phi_r_cut, idx_i, idx_j, pair_mask, W_Q, W_K, mlp_W0, mlp_b0, mlp_W1, mlp_b1):
    raise NotImplementedError("write your pallas kernel here")


# speedup vs baseline: 8.3213x; 8.3213x over previous
import functools
import math

import jax
import jax.numpy as jnp
from jax import lax
from jax.experimental import pallas as pl
from jax.experimental.pallas import tpu as pltpu
from jax.experimental.pallas import tpu_sc as plsc

_NC = 2
_NS = 16
_NW = _NC * _NS

_T = 64
_B = 512

_INTERPRET = False


def _sc_gather_rows(table, idx):
    P = idx.shape[0]
    D = table.shape[1]
    per_w = P // _NW
    CH = 128
    n_full = per_w // CH
    tail = per_w - n_full * CH
    mesh = plsc.VectorSubcoreMesh(core_axis_name="c", subcore_axis_name="s",
                                  num_cores=_NC, num_subcores=_NS)

    @functools.partial(
        pl.kernel,
        out_type=jax.ShapeDtypeStruct((P, D), jnp.float32),
        mesh=mesh,
        scratch_types=[
            pltpu.VMEM((per_w,), jnp.int32),
            pltpu.VMEM((CH, D), jnp.float32),
            pltpu.SemaphoreType.DMA,
        ],
    )
    def gather_k(table_hbm, idx_hbm, out_hbm, idx_v, rows_v, sem):
        wid = lax.axis_index("s") * _NC + lax.axis_index("c")
        base = wid * per_w
        pltpu.sync_copy(idx_hbm.at[pl.ds(base, per_w)], idx_v)

        def chunk(c, carry):
            off = c * CH
            pltpu.async_copy(table_hbm.at[idx_v.at[pl.ds(off, CH)]],
                             rows_v, sem).wait()
            pltpu.sync_copy(rows_v, out_hbm.at[pl.ds(base + off, CH)])
            return carry

        lax.fori_loop(0, n_full, chunk, 0)
        if tail:
            off = n_full * CH
            pltpu.async_copy(table_hbm.at[idx_v.at[pl.ds(off, tail)]],
                             rows_v.at[pl.ds(0, tail)], sem).wait()
            pltpu.sync_copy(rows_v.at[pl.ds(0, tail)],
                            out_hbm.at[pl.ds(base + off, tail)])

    return gather_k(table, idx)


def _fused_body(n_nodes, scale, tt_ref, bb_ref, ff_ref, x_ref, rbf_ref,
                aux_ref, idx_ref, xj_ref, wq_ref, wk_ref, w0_ref, b0_ref,
                w1_ref, b1_ref, rsum_ref, out_ref):
    H = lax.Precision.HIGHEST
    g = pl.program_id(0)
    t = tt_ref[g]
    nf = jnp.float32(n_nodes)

    @pl.when(ff_ref[g] == 1)
    def _zero():
        out_ref[...] = jnp.zeros_like(out_ref)

    rbf = rbf_ref[...]
    aux = aux_ref[...]
    idx = idx_ref[...][:, 0:1]
    xj = xj_ref[...]
    sph = aux[:, 0:4]
    phi = aux[:, 4:5]
    msk = aux[:, 5:6]

    h = jnp.dot(rbf, w0_ref[...], precision=H) + b0_ref[0:1, :]
    h = h * jax.nn.sigmoid(h)
    w = jnp.dot(h, w1_ref[...], precision=H) + b1_ref[0:1, :]
    wij = w * phi

    il = idx - t * _T
    S = (il == lax.broadcasted_iota(jnp.int32, (_B, _T), 1)).astype(jnp.float32)
    xi = jnp.dot(S, x_ref[...], precision=H)

    q = jnp.dot(xi, wq_ref[...], precision=H)
    kk = jnp.dot(xj, wk_ref[...], precision=H)
    ok = msk != 0
    q = jnp.where(ok, q * msk, nf)
    kk = jnp.where(ok, kk * msk, nf)
    prod = q * kk * wij
    alpha8 = jnp.dot(prod, rsum_ref[...], precision=H) * jnp.float32(scale)
    a0 = alpha8[:, 0:4]
    a1 = alpha8[:, 4:8]

    xjm = jnp.where(ok, xj * msk, nf).reshape(_B, 4, 32)
    zs = []
    for m in range(4):
        am = a0 if m == 0 else a1
        zm = (am[:, :, None] * xjm).reshape(_B, 128) * sph[:, m:m + 1]
        zs.append(zm)
    z = jnp.concatenate(zs, axis=1)

    out_ref[...] += lax.dot_general(S, z, (((0,), (0,)), ((), ())), precision=H)


def kernel(x, rbf_ij, sph_ij, phi_r_cut, idx_i, idx_j, pair_mask,
           W_Q, W_K, mlp_W0, mlp_b0, mlp_W1, mlp_b1):
    n, F = x.shape
    P = idx_i.shape[0]
    n_head = W_Q.shape[1]
    n_deg = W_Q.shape[0]
    m_tot = sph_ij.shape[1]
    d_h = F // n_head
    scale = 1.0 / math.sqrt(F / n_head)

    NT = -(-n // _T)
    NB = -(-P // _B)
    NG = NB + NT
    n_rows = NT * _T
    Ppad = (NB + 1) * _B

    idx_i = idx_i.astype(jnp.int32)
    idx_j = idx_j.astype(jnp.int32)

    xj_g = _sc_gather_rows(x, idx_j)

    aux = jnp.concatenate(
        [sph_ij, phi_r_cut[:, None], pair_mask[:, None],
         jnp.zeros((P, 2), jnp.float32)], axis=1)

    pad = Ppad - P
    rbf_p = jnp.concatenate(
        [rbf_ij, jnp.zeros((pad, rbf_ij.shape[1]), jnp.float32)], 0)
    aux_p = jnp.concatenate([aux, jnp.zeros((pad, 8), jnp.float32)], 0)
    idx8 = jnp.tile(idx_i[:, None], (1, 8))
    idx8_p = jnp.concatenate([idx8, jnp.full((pad, 8), n_rows, jnp.int32)], 0)
    xj_p = jnp.concatenate([xj_g, jnp.zeros((pad, F), jnp.float32)], 0)
    x_pad = jnp.concatenate([x, jnp.zeros((n_rows - n, F), jnp.float32)], 0)

    starts = jnp.searchsorted(
        idx_i, jnp.arange(NT + 1, dtype=jnp.int32) * _T).astype(jnp.int32)
    s_lo = starts[:-1]
    s_hi = starts[1:]
    nonempty = s_hi > s_lo
    fb = s_lo // _B
    lb = jnp.where(nonempty, (s_hi - 1) // _B, fb)
    cnt = jnp.where(nonempty, lb - fb + 1, 1)
    fbp = jnp.where(nonempty, fb, NB)
    o = jnp.concatenate([jnp.zeros((1,), jnp.int32),
                         jnp.cumsum(cnt, dtype=jnp.int32)])
    tot = o[NT]
    gg = jnp.arange(NG, dtype=jnp.int32)
    tt = jnp.searchsorted(o, gg, side='right').astype(jnp.int32) - 1
    real = gg < tot
    tt = jnp.where(real, jnp.clip(tt, 0, NT - 1), NT - 1)
    bb = jnp.where(real, fbp[tt] + (gg - o[tt]), NB)
    bb = jnp.clip(bb, 0, NB)
    ff = jnp.concatenate([jnp.ones((1,), jnp.int32),
                          (tt[1:] != tt[:-1]).astype(jnp.int32)])

    eye = jnp.eye(n_head, dtype=jnp.float32)
    WQb = jnp.einsum('hg,dgab->hbdga', eye, W_Q).reshape(F, n_deg * F)
    WKb = jnp.einsum('hg,dgab->hbdga', eye, W_K).reshape(F, n_deg * F)
    rsum = (jnp.arange(n_deg * F)[:, None] // d_h
            == jnp.arange(n_deg * n_head)[None, :]).astype(jnp.float32)
    b0_t = jnp.tile(mlp_b0[None, :], (8, 1))
    b1_t = jnp.tile(mlp_b1[None, :], (8, 1))

    grid_spec = pltpu.PrefetchScalarGridSpec(
        num_scalar_prefetch=3,
        grid=(NG,),
        in_specs=[
            pl.BlockSpec((_T, F), lambda g, tt, bb, ff: (tt[g], 0)),
            pl.BlockSpec((_B, 32), lambda g, tt, bb, ff: (bb[g], 0)),
            pl.BlockSpec((_B, 8), lambda g, tt, bb, ff: (bb[g], 0)),
            pl.BlockSpec((_B, 8), lambda g, tt, bb, ff: (bb[g], 0)),
            pl.BlockSpec((_B, F), lambda g, tt, bb, ff: (bb[g], 0)),
            pl.BlockSpec((F, n_deg * F), lambda g, tt, bb, ff: (0, 0)),
            pl.BlockSpec((F, n_deg * F), lambda g, tt, bb, ff: (0, 0)),
            pl.BlockSpec((32, 64), lambda g, tt, bb, ff: (0, 0)),
            pl.BlockSpec((8, 64), lambda g, tt, bb, ff: (0, 0)),
            pl.BlockSpec((64, n_deg * F), lambda g, tt, bb, ff: (0, 0)),
            pl.BlockSpec((8, n_deg * F), lambda g, tt, bb, ff: (0, 0)),
            pl.BlockSpec((n_deg * F, n_deg * n_head),
                         lambda g, tt, bb, ff: (0, 0)),
        ],
        out_specs=pl.BlockSpec((_T, m_tot * F), lambda g, tt, bb, ff: (tt[g], 0)),
    )

    out_pad = pl.pallas_call(
        functools.partial(_fused_body, n, scale),
        grid_spec=grid_spec,
        out_shape=jax.ShapeDtypeStruct((n_rows, m_tot * F), jnp.float32),
        interpret=_INTERPRET,
    )(tt, bb, ff, x_pad, rbf_p, aux_p, idx8_p, xj_p, WQb, WKb,
      mlp_W0, b0_t, mlp_W1, b1_t, rsum)

    return out_pad[:n].reshape(n, m_tot, F)
